# Initial kernel scaffold; baseline (speedup 1.0000x reference)
import jax
import jax.numpy as jnp
from jax.experimental import pallas as pl


def kernel(V_embed, E_embed, X, x_mask, chain_idx, W, b):
    raise NotImplementedError("write your pallas kernel here")


# speedup vs baseline: 6.3681x; 6.3681x over previous
import functools

import jax
import jax.numpy as jnp
from jax import lax
from jax.experimental import pallas as pl
from jax.experimental.pallas import tpu as pltpu
from jax.experimental.pallas import tpu_sc as plsc

KNN = 30
ODIM = 400

SC_CORES = 2
SC_SUBCORES = 16
NWORKERS = SC_CORES * SC_SUBCORES


def _topk_body(ca_col_ref, ca_row_ref, eidx_ref):
    L = ca_col_ref.shape[1]
    acc = None
    for d in range(3):
        col = ca_col_ref[0, :, d : d + 1]
        row = ca_row_ref[0, d : d + 1, :]
        diff = col - row
        sq = diff * diff
        acc = sq if acc is None else acc + sq
    dist = jnp.sqrt(acc + 1e-6)

    iota_j = lax.broadcasted_iota(jnp.int32, (L, L), 1)
    vals = dist
    for k in range(KNN):
        m = jnp.min(vals, axis=1, keepdims=True)
        sel = vals == m
        idx = jnp.min(jnp.where(sel, iota_j, L), axis=1, keepdims=True)
        eidx_ref[0, :, k : k + 1] = idx
        vals = jnp.where(iota_j == idx, jnp.inf, vals)


def _run_topk(Ca):
    B, L, _ = Ca.shape
    Ca_row = jnp.transpose(Ca, (0, 2, 1))
    return pl.pallas_call(
        _topk_body,
        grid=(B,),
        in_specs=[
            pl.BlockSpec((1, L, 3), lambda b: (b, 0, 0)),
            pl.BlockSpec((1, 3, L), lambda b: (b, 0, 0)),
        ],
        out_specs=pl.BlockSpec((1, L, KNN), lambda b: (b, 0, 0)),
        out_shape=jax.ShapeDtypeStruct((B, L, KNN), jnp.int32),
    )(Ca, Ca_row)


def _sc_gather(E_flat, F_idx, V_flat, G_idx, G0_idx):
    N = F_idx.shape[0]
    C = E_flat.shape[1]
    BL = G0_idx.shape[0]
    per_w = N // NWORKERS
    CH = 120
    n_ch = per_w // CH
    assert per_w % CH == 0 and per_w % 8 == 0
    per_w0 = BL // NWORKERS
    assert per_w0 % 8 == 0

    mesh = plsc.VectorSubcoreMesh(
        core_axis_name="c",
        subcore_axis_name="s",
        num_cores=SC_CORES,
        num_subcores=SC_SUBCORES,
    )

    @functools.partial(
        pl.kernel,
        mesh=mesh,
        out_type=(
            jax.ShapeDtypeStruct((N, C), jnp.float32),
            jax.ShapeDtypeStruct((N, C), jnp.float32),
            jax.ShapeDtypeStruct((BL, C), jnp.float32),
        ),
        scratch_types=[
            pltpu.VMEM((CH,), jnp.int32),
            pltpu.VMEM((CH, C), jnp.float32),
            pltpu.VMEM((per_w0,), jnp.int32),
            pltpu.VMEM((per_w0, C), jnp.float32),
            pltpu.SemaphoreType.DMA,
        ],
    )
    def gather_kernel(e_hbm, f_hbm, v_hbm, g_hbm, g0_hbm,
                      eg_out, vg_out, v0_out,
                      idx_v, rows_v, idx0_v, rows0_v, sem):
        wid = lax.axis_index("s") * SC_CORES + lax.axis_index("c")
        base_w = wid * per_w

        def do_chunks(table, idx_hbm, out_hbm):
            def body(c, _):
                base = base_w + c * CH
                pltpu.sync_copy(idx_hbm.at[pl.ds(base, CH)], idx_v)
                pltpu.async_copy(table.at[idx_v], rows_v, sem).wait()
                pltpu.sync_copy(rows_v, out_hbm.at[pl.ds(base, CH)])
                return ()

            lax.fori_loop(0, n_ch, body, (), unroll=False)

        do_chunks(e_hbm, f_hbm, eg_out)
        do_chunks(v_hbm, g_hbm, vg_out)

        base0 = wid * per_w0
        pltpu.sync_copy(g0_hbm.at[pl.ds(base0, per_w0)], idx0_v)
        pltpu.async_copy(v_hbm.at[idx0_v], rows0_v, sem).wait()
        pltpu.sync_copy(rows0_v, v0_out.at[pl.ds(base0, per_w0)])

    return gather_kernel(E_flat, F_idx, V_flat, G_idx, G0_idx)


def _linear_body(eg_ref, vg_ref, v0_ref, w_ref, b_ref, out_ref):
    C = eg_ref.shape[1]
    RI = v0_ref.shape[0]
    R = eg_ref.shape[0]
    w1 = w_ref[0:C, :]
    w2 = w_ref[C : 2 * C, :]
    w3 = w_ref[2 * C : 3 * C, :]
    m0 = jnp.dot(v0_ref[...], w1, preferred_element_type=jnp.float32)
    r = lax.broadcasted_iota(jnp.int32, (R, RI), 0)
    c = lax.broadcasted_iota(jnp.int32, (R, RI), 1)
    S = (r // KNN == c).astype(jnp.float32)
    acc = jnp.dot(eg_ref[...], w3, preferred_element_type=jnp.float32)
    acc = acc + jnp.dot(vg_ref[...], w2, preferred_element_type=jnp.float32)
    acc = acc + jnp.dot(S, m0, preferred_element_type=jnp.float32)
    out_ref[...] = acc + b_ref[...]


def _run_linear(Eg, Vg, V0g, W, b):
    N, C = Eg.shape
    BL = V0g.shape[0]
    RI = 16
    R = RI * KNN
    grid = N // R
    assert N % R == 0
    return pl.pallas_call(
        _linear_body,
        grid=(grid,),
        in_specs=[
            pl.BlockSpec((R, C), lambda g: (g, 0)),
            pl.BlockSpec((R, C), lambda g: (g, 0)),
            pl.BlockSpec((RI, C), lambda g: (g, 0)),
            pl.BlockSpec((3 * C, ODIM), lambda g: (0, 0)),
            pl.BlockSpec((1, ODIM), lambda g: (0, 0)),
        ],
        out_specs=pl.BlockSpec((R, ODIM), lambda g: (g, 0)),
        out_shape=jax.ShapeDtypeStruct((N, ODIM), jnp.float32),
    )(Eg, Vg, V0g, W, b.reshape(1, ODIM))


def kernel(V_embed, E_embed, X, x_mask, chain_idx, W, b):
    B, L, C = V_embed.shape
    Ca = X[:, :, 1, :]

    E_idx = _run_topk(Ca)

    row_base = (jnp.arange(B, dtype=jnp.int32)[:, None] * L
                + jnp.arange(L, dtype=jnp.int32)[None, :])
    F_idx = (row_base[..., None] * L + E_idx).reshape(-1)
    G_idx = (jnp.arange(B, dtype=jnp.int32)[:, None, None] * L
             + E_idx).reshape(-1)
    G0_idx = (jnp.arange(B, dtype=jnp.int32)[:, None] * L
              + E_idx[:, :, 0]).reshape(-1)

    E_flat = E_embed.reshape(B * L * L, C)
    V_flat = V_embed.reshape(B * L, C)

    Eg, Vg, V0g = _sc_gather(E_flat, F_idx, V_flat, G_idx, G0_idx)

    out = _run_linear(Eg, Vg, V0g, W, b)
    h_EV = out.reshape(B, L, KNN, ODIM)
    return (h_EV, E_idx)


# speedup vs baseline: 11.2807x; 1.7714x over previous
import functools

import jax
import jax.numpy as jnp
from jax import lax
from jax.experimental import pallas as pl
from jax.experimental.pallas import tpu as pltpu
from jax.experimental.pallas import tpu_sc as plsc

KNN = 30
ODIM = 400

SC_CORES = 2
SC_SUBCORES = 16
NWORKERS = SC_CORES * SC_SUBCORES


def _topk_body(ca_col_ref, ca_row_ref, eidx_ref):
    L = ca_col_ref.shape[1]
    acc = None
    for d in range(3):
        col = ca_col_ref[0, :, d : d + 1]
        row = ca_row_ref[0, d : d + 1, :]
        diff = col - row
        sq = diff * diff
        acc = sq if acc is None else acc + sq
    dist = jnp.sqrt(acc + 1e-6)

    iota_j = lax.broadcasted_iota(jnp.int32, (L, L), 1)
    vals = dist
    for k in range(KNN):
        m = jnp.min(vals, axis=1, keepdims=True)
        sel = vals == m
        idx = jnp.min(jnp.where(sel, iota_j, L), axis=1, keepdims=True)
        eidx_ref[0, :, k : k + 1] = idx
        vals = jnp.where(iota_j == idx, jnp.inf, vals)


def _run_topk(Ca):
    B, L, _ = Ca.shape
    Ca_row = jnp.transpose(Ca, (0, 2, 1))
    return pl.pallas_call(
        _topk_body,
        grid=(B,),
        in_specs=[
            pl.BlockSpec((1, L, 3), lambda b: (b, 0, 0)),
            pl.BlockSpec((1, 3, L), lambda b: (b, 0, 0)),
        ],
        out_specs=pl.BlockSpec((1, L, KNN), lambda b: (b, 0, 0)),
        out_shape=jax.ShapeDtypeStruct((B, L, KNN), jnp.int32),
    )(Ca, Ca_row)


def _sc_gather(E_flat, F_idx, V_flat, G_idx, G0_idx):
    N = F_idx.shape[0]
    C = E_flat.shape[1]
    BL = G0_idx.shape[0]
    per_w = N // NWORKERS
    CH = 120
    n_ch = per_w // CH
    assert per_w % CH == 0
    per_w0 = BL // NWORKERS
    assert per_w0 % 8 == 0

    mesh = plsc.VectorSubcoreMesh(
        core_axis_name="c",
        subcore_axis_name="s",
        num_cores=SC_CORES,
        num_subcores=SC_SUBCORES,
    )

    @functools.partial(
        pl.kernel,
        mesh=mesh,
        out_type=(
            jax.ShapeDtypeStruct((N, C), jnp.float32),
            jax.ShapeDtypeStruct((N, C), jnp.float32),
            jax.ShapeDtypeStruct((BL, C), jnp.float32),
        ),
        scratch_types=[
            pltpu.VMEM((CH,), jnp.int32),
            pltpu.VMEM((CH,), jnp.int32),
            pltpu.VMEM((CH, C), jnp.float32),
            pltpu.VMEM((CH, C), jnp.float32),
            pltpu.VMEM((per_w0,), jnp.int32),
            pltpu.VMEM((per_w0, C), jnp.float32),
            pltpu.SemaphoreType.DMA,
            pltpu.SemaphoreType.DMA,
            pltpu.SemaphoreType.DMA,
            pltpu.SemaphoreType.DMA,
        ],
    )
    def gather_kernel(e_hbm, f_hbm, v_hbm, g_hbm, g0_hbm,
                      eg_out, vg_out, v0_out,
                      idx_a, idx_b, rows_a, rows_b, idx0_v, rows0_v,
                      gsem_a, gsem_b, ssem_a, ssem_b):
        wid = lax.axis_index("s") * SC_CORES + lax.axis_index("c")
        base_w = wid * per_w
        idx_bufs = (idx_a, idx_b)
        row_bufs = (rows_a, rows_b)
        gsems = (gsem_a, gsem_b)
        ssems = (ssem_a, ssem_b)

        def do_chunks(table, idx_hbm, out_hbm):
            pltpu.sync_copy(idx_hbm.at[pl.ds(base_w, CH)], idx_bufs[0])
            pltpu.async_copy(table.at[idx_bufs[0]], row_bufs[0], gsems[0])
            for c in range(n_ch):
                p = c % 2
                q = 1 - p
                pltpu.make_async_copy(
                    table.at[idx_bufs[p]], row_bufs[p], gsems[p]
                ).wait()
                if c >= 1:
                    pltpu.make_async_copy(
                        row_bufs[q], out_hbm.at[pl.ds(0, CH)], ssems[q]
                    ).wait()
                if c + 1 < n_ch:
                    pltpu.sync_copy(
                        idx_hbm.at[pl.ds(base_w + (c + 1) * CH, CH)],
                        idx_bufs[q],
                    )
                    pltpu.async_copy(table.at[idx_bufs[q]], row_bufs[q],
                                     gsems[q])
                pltpu.async_copy(
                    row_bufs[p], out_hbm.at[pl.ds(base_w + c * CH, CH)],
                    ssems[p],
                )
            last = (n_ch - 1) % 2
            pltpu.make_async_copy(
                row_bufs[last], out_hbm.at[pl.ds(0, CH)], ssems[last]
            ).wait()

        do_chunks(e_hbm, f_hbm, eg_out)
        do_chunks(v_hbm, g_hbm, vg_out)

        base0 = wid * per_w0
        pltpu.sync_copy(g0_hbm.at[pl.ds(base0, per_w0)], idx0_v)
        pltpu.async_copy(v_hbm.at[idx0_v], rows0_v, gsem_a).wait()
        pltpu.sync_copy(rows0_v, v0_out.at[pl.ds(base0, per_w0)])

    return gather_kernel(E_flat, F_idx, V_flat, G_idx, G0_idx)


_DN_T = (((0,), (1,)), ((), ()))


def _linear_body(eg_ref, vg_ref, v0_ref, w_ref, b_ref, out_ref, m0t_s):
    C = eg_ref.shape[1]
    k = pl.program_id(1)

    @pl.when(k == 0)
    def _():
        m0t_s[...] = lax.dot_general(
            w_ref[0:C, :], v0_ref[...], _DN_T,
            preferred_element_type=jnp.float32,
        ) + b_ref[...]

    w2 = w_ref[C : 2 * C, :]
    w3 = w_ref[2 * C : 3 * C, :]
    acc = lax.dot_general(w2, vg_ref[...], _DN_T,
                          preferred_element_type=jnp.float32)
    acc = acc + lax.dot_general(w3, eg_ref[...], _DN_T,
                                preferred_element_type=jnp.float32)
    out_ref[0, 0] = acc + m0t_s[...]


def _run_linear(Eg, Vg, V0g, W, b, B, L):
    C = Eg.shape[1]
    P = pl.pallas_call(
        _linear_body,
        grid=(B, KNN),
        in_specs=[
            pl.BlockSpec((L, C), lambda b_, k: (b_ * KNN + k, 0)),
            pl.BlockSpec((L, C), lambda b_, k: (b_ * KNN + k, 0)),
            pl.BlockSpec((L, C), lambda b_, k: (b_, 0)),
            pl.BlockSpec((3 * C, ODIM), lambda b_, k: (0, 0)),
            pl.BlockSpec((ODIM, 1), lambda b_, k: (0, 0)),
        ],
        out_specs=pl.BlockSpec((1, 1, ODIM, L), lambda b_, k: (b_, k, 0, 0)),
        out_shape=jax.ShapeDtypeStruct((B, KNN, ODIM, L), jnp.float32),
        scratch_shapes=[pltpu.VMEM((ODIM, L), jnp.float32)],
    )(Eg, Vg, V0g, W, b.reshape(ODIM, 1))
    return jnp.transpose(P, (0, 3, 1, 2))


def kernel(V_embed, E_embed, X, x_mask, chain_idx, W, b):
    B, L, C = V_embed.shape
    Ca = X[:, :, 1, :]

    E_idx = _run_topk(Ca)

    E_idx_t = jnp.transpose(E_idx, (0, 2, 1))
    row_base = (jnp.arange(B, dtype=jnp.int32)[:, None, None] * L
                + jnp.arange(L, dtype=jnp.int32)[None, None, :])
    F_idx = (row_base * L + E_idx_t).reshape(-1)
    G_idx = (jnp.arange(B, dtype=jnp.int32)[:, None, None] * L
             + E_idx_t).reshape(-1)
    G0_idx = (jnp.arange(B, dtype=jnp.int32)[:, None] * L
              + E_idx[:, :, 0]).reshape(-1)

    E_flat = E_embed.reshape(B * L * L, C)
    V_flat = V_embed.reshape(B * L, C)

    Eg, Vg, V0g = _sc_gather(E_flat, F_idx, V_flat, G_idx, G0_idx)

    h_EV = _run_linear(Eg, Vg, V0g, W, b, B, L)
    return (h_EV, E_idx)
